# baseline (speedup 1.0000x reference)
import functools

import jax
import jax.numpy as jnp
from jax import lax
from jax.experimental import pallas as pl
from jax.experimental.pallas import tpu as pltpu
from jax.experimental.pallas import tpu_sc as plsc

NC, NS, L = 2, 16, 16
NW = NC * NS
CH = 128


def _wid():
    return lax.axis_index("s") * NC + lax.axis_index("c")


@functools.lru_cache(maxsize=None)
def _make_gather(MH, B, D2):
    b_per_w = B // NW
    nch = b_per_w // CH
    mesh = plsc.VectorSubcoreMesh(core_axis_name="c", subcore_axis_name="s")

    @functools.partial(
        pl.kernel, mesh=mesh,
        out_type=(jax.ShapeDtypeStruct((B, D2), jnp.float32),
                  jax.ShapeDtypeStruct((B, D2), jnp.float32)),
        compiler_params=pltpu.CompilerParams(needs_layout_passes=False),
        scratch_types=[pltpu.VMEM((b_per_w,), jnp.int32),
                       pltpu.VMEM((b_per_w, D2), jnp.float32),
                       pltpu.SemaphoreType.DMA],
    )
    def k(src_tab, dst_tab, src_pidx, dst_pidx, out_s, out_d, idx_v, rows_v, sem):
        base = _wid() * b_per_w
        for tab, idx, out in ((src_tab, src_pidx, out_s), (dst_tab, dst_pidx, out_d)):
            pltpu.sync_copy(idx.at[pl.ds(base, b_per_w)], idx_v)
            descs = []
            for j in range(nch):
                descs.append(pltpu.async_copy(
                    tab.at[idx_v.at[pl.ds(j * CH, CH)]],
                    rows_v.at[pl.ds(j * CH, CH)], sem))
            for d in descs:
                d.wait()
            pltpu.sync_copy(rows_v, out.at[pl.ds(base, b_per_w)])

    return k


@functools.lru_cache(maxsize=None)
def _make_mlp(B, DN, DG, DH):
    BLK = 2048
    grid = B // BLK
    D2 = 2 * DN

    def body(ids_s, ids_d, gs, ns_, gd, nd_, wsr, wsn, wsl, bsr, bsn, bsl,
             wdr, wdn, wdl, bdr, bdn, bdl, os_, od_):
        f32 = jnp.float32
        i = pl.program_id(0)
        for ids, g2, n, wr, wn, wl, br, bn, bl, out in (
                (ids_s, gs, ns_, wsr, wsn, wsl, bsr, bsn, bsl, os_),
                (ids_d, gd, nd_, wdr, wdn, wdl, bdr, bdn, bdl, od_)):
            par = (ids[pl.ds(i * BLK, BLK)] & 1)[:, None]
            g2v = g2[...]
            g = jnp.where(par == 1, g2v[:, DN:2 * DN], g2v[:, 0:DN])
            wl_top = wl[0:DH, :]
            wl_bot = wl[DH:2 * DH, :]
            Wg = jnp.dot(wr[...], wl_top, preferred_element_type=f32)
            Wn = jnp.dot(wn[...], wl_bot, preferred_element_type=f32)
            b_eff = (jnp.dot(br[...], wl_top, preferred_element_type=f32)
                     + jnp.dot(bn[...], wl_bot, preferred_element_type=f32)
                     + bl[...])
            new = (jnp.dot(g, Wg, preferred_element_type=f32)
                   + jnp.dot(n[...], Wn, preferred_element_type=f32)
                   + b_eff)
            out[...] = jnp.concatenate([new, new], axis=1)

    whole = lambda n: pl.BlockSpec((n,), lambda i: (0,))
    full = lambda shape: pl.BlockSpec(shape, lambda i: (0,) * len(shape))
    row = lambda d: pl.BlockSpec((BLK, d), lambda i: (i, 0))
    return pl.pallas_call(
        body,
        grid=(grid,),
        in_specs=[whole(B), whole(B),
                  row(D2), row(DG), row(D2), row(DG),
                  full((DN, DH)), full((DG, DH)), full((2 * DH, DN)),
                  full((1, DH)), full((1, DH)), full((1, DN)),
                  full((DN, DH)), full((DG, DH)), full((2 * DH, DN)),
                  full((1, DH)), full((1, DH)), full((1, DN))],
        out_specs=[row(D2), row(D2)],
        out_shape=(jax.ShapeDtypeStruct((B, D2), jnp.float32),
                   jax.ShapeDtypeStruct((B, D2), jnp.float32)),
    )


@functools.lru_cache(maxsize=None)
def _make_update(MH, B, D2):
    SLABH = (MH // NW) & ~7
    REM = MH - SLABH * NW
    assert REM % 8 == 0 and REM >= 0
    NIT = B // L
    KROWS = 33
    QW = D2 // (2 * L)
    CPR = 128
    NFULL = SLABH // CPR
    TAILR = SLABH - NFULL * CPR
    NG = NFULL // 4
    NGR = NFULL - NG * 4
    mesh = plsc.VectorSubcoreMesh(core_axis_name="c", subcore_axis_name="s")

    @functools.partial(
        pl.kernel, mesh=mesh,
        out_type=(jax.ShapeDtypeStruct((MH, D2), jnp.float32),
                  jax.ShapeDtypeStruct((MH, D2), jnp.float32)),
        compiler_params=pltpu.CompilerParams(needs_layout_passes=False),
        scratch_types=[pltpu.VMEM((B,), jnp.int32),
                       pltpu.VMEM((KROWS, CH), jnp.int32),
                       pltpu.VMEM((KROWS, CH), jnp.int32),
                       pltpu.VMEM((KROWS * CH,), jnp.int32),
                       pltpu.VMEM((KROWS * CH,), jnp.int32),
                       pltpu.VMEM((CH, D2), jnp.float32),
                       pltpu.VMEM((CH, D2), jnp.float32),
                       pltpu.VMEM((CPR, D2), jnp.float32),
                       pltpu.VMEM((CPR, D2), jnp.float32),
                       pltpu.SemaphoreType.DMA,
                       pltpu.SemaphoreType.DMA],
    )
    def k(src_tab, dst_tab, src_idx, dst_idx, new_src, new_dst,
          out_s, out_d, ids_v, pair2d_e, pair2d_o, pos_e, pos_o,
          rows_st, vals_st, cb2, cb3, sem_c, sem_rw):
        wid = _wid()
        plo = wid * SLABH
        lo = plo * 2
        hi = jnp.where(wid == NW - 1, 2 * MH, lo + SLABH * 2)
        cbufs = (rows_st, vals_st, cb2, cb3)

        def copy_slab(tab, out):
            def group(g, _):
                base = plo + g * (4 * CPR)
                rds = [pltpu.async_copy(tab.at[pl.ds(base + i * CPR, CPR)],
                                        cbufs[i], sem_c) for i in range(4)]
                for r in rds:
                    r.wait()
                wrs = [pltpu.async_copy(
                    cbufs[i], out.at[pl.ds(base + i * CPR, CPR)], sem_c)
                    for i in range(4)]
                for w in wrs:
                    w.wait()
                return 0

            lax.fori_loop(0, NG, group, 0)
            base = plo + NG * (4 * CPR)
            rds, wrs, exts = [], [], []
            for i in range(NGR):
                rds.append((pltpu.async_copy(
                    tab.at[pl.ds(base + i * CPR, CPR)], cbufs[i], sem_c),
                    cbufs[i], base + i * CPR, CPR))
            if TAILR:
                t = base + NGR * CPR
                rds.append((pltpu.async_copy(
                    tab.at[pl.ds(t, TAILR)], cbufs[3].at[pl.ds(0, TAILR)],
                    sem_c), cbufs[3].at[pl.ds(0, TAILR)], t, TAILR))
            for rd, buf, off, nr in rds:
                rd.wait()
            for rd, buf, off, nr in rds:
                wrs.append(pltpu.async_copy(buf, out.at[pl.ds(off, nr)],
                                            sem_c))
            for w in wrs:
                w.wait()

        def copy_rem(tab, out):
            t0 = NW * SLABH
            pltpu.async_copy(tab.at[pl.ds(t0, REM)],
                             cb2.at[pl.ds(0, REM)], sem_c).wait()
            pltpu.async_copy(cb2.at[pl.ds(0, REM)],
                             out.at[pl.ds(t0, REM)], sem_c).wait()

        iota = lax.iota(jnp.int32, L)
        one = jnp.int32(1)
        zero = jnp.int32(0)

        def splat_last(x):
            return plsc.cummax(lax.rev(plsc.cummax(x), (0,)))

        def scan_ids(idx_hbm):
            pltpu.sync_copy(idx_hbm, ids_v)

            def step(i, carry):
                cnt_e, cnt_o = carry
                v = ids_v[pl.ds(i * L, L)]
                posv = i * L + iota
                inb = (v >= lo) & (v < hi)
                pair = v >> 1
                odd = (v & 1) == 1
                m_e = inb & jnp.logical_not(odd)
                m_o = inb & odd
                for m, cnt, pair2d, pos in ((m_e, cnt_e, pair2d_e, pos_e),
                                            (m_o, cnt_o, pair2d_o, pos_o)):
                    incl = plsc.cumsum(jnp.where(m, one, zero))
                    tgt = jnp.maximum(cnt + incl - 1, 0)
                    plsc.store_scatter(pair2d, [tgt >> 7, tgt & (CH - 1)],
                                       pair, mask=m)
                    plsc.store_scatter(pos, [tgt], posv, mask=m)
                return (cnt_e + plsc.all_reduce_population_count(m_e),
                        cnt_o + plsc.all_reduce_population_count(m_o))

            zeros = jnp.zeros((L,), jnp.int32)
            cnt_e, cnt_o = lax.fori_loop(0, NIT, step, (zeros, zeros))

            def finish(cnt_v, pair2d, pos):
                n = cnt_v[L - 1]
                n_pad = ((n + CH - 1) // CH) * CH
                j = jnp.maximum(n - 1, 0)
                vrow = pair2d[j >> 7, pl.ds((j & (CH - 1)) & ~(L - 1), L)]
                vpos = pos[pl.ds(j & ~(L - 1), L)]
                lane = j & (L - 1)
                pad_pair = splat_last(jnp.where(iota == lane, vrow, -1))
                pad_pos = splat_last(jnp.where(iota == lane, vpos, -1))

                def pad_step(t, _):
                    p = n + t * L + iota
                    plsc.store_scatter(pair2d, [p >> 7, p & (CH - 1)], pad_pair)
                    plsc.store_scatter(pos, [p], pad_pos)
                    return 0

                lax.fori_loop(0, (n_pad - n + L - 1) // L, pad_step, 0)
                return n_pad

            return finish(cnt_e, pair2d_e, pos_e), finish(cnt_o, pair2d_o, pos_o)

        def apply_pass(new_hbm, out_hbm, pair2d, pos, n_pad, off):
            def chunk(c, _):
                pltpu.async_copy(out_hbm.at[pair2d.at[c]], rows_st,
                                 sem_rw).wait()
                pltpu.async_copy(new_hbm.at[pos.at[pl.ds(c * CH, CH)]],
                                 vals_st, sem_rw).wait()

                def merge(r, _):
                    for q in range(QW):
                        rows_st[r, pl.ds(off + q * L, L)] = \
                            vals_st[r, pl.ds(q * L, L)]
                    return 0

                lax.fori_loop(0, CH, merge, 0)
                pltpu.async_copy(rows_st, out_hbm.at[pair2d.at[c]],
                                 sem_rw).wait()
                return 0

            lax.fori_loop(0, n_pad // CH, chunk, 0)

        ne, no = scan_ids(src_idx)
        copy_slab(src_tab, out_s)
        copy_slab(dst_tab, out_d)
        if REM:
            @pl.when(wid == NW - 1)
            def _():
                copy_rem(src_tab, out_s)
                copy_rem(dst_tab, out_d)
        apply_pass(new_src, out_s, pair2d_e, pos_e, ne, 0)
        apply_pass(new_src, out_s, pair2d_o, pos_o, no, D2 // 2)
        ne, no = scan_ids(dst_idx)
        apply_pass(new_dst, out_d, pair2d_e, pos_e, ne, 0)
        apply_pass(new_dst, out_d, pair2d_o, pos_o, no, D2 // 2)

    return k


def kernel(src_node_ids, dst_node_ids, src_previous_embedding,
           dst_previous_embedding, batch_src_neighbor_embedding,
           batch_dst_neighbor_embedding, W_sn, b_sn, W_sr, b_sr, W_sl, b_sl,
           W_dn, b_dn, W_dr, b_dr, W_dl, b_dl):
    M, DN = src_previous_embedding.shape
    B = src_node_ids.shape[0]
    DG = batch_src_neighbor_embedding.shape[1]
    DH = W_sn.shape[1]
    MH, D2 = M // 2, 2 * DN

    sidx = src_node_ids.astype(jnp.int32)
    didx = dst_node_ids.astype(jnp.int32)
    th_src = src_previous_embedding.reshape(MH, D2)
    th_dst = dst_previous_embedding.reshape(MH, D2)

    g2_src, g2_dst = _make_gather(MH, B, D2)(th_src, th_dst,
                                             sidx >> 1, didx >> 1)

    new_src, new_dst = _make_mlp(B, DN, DG, DH)(
        sidx, didx, g2_src, batch_src_neighbor_embedding, g2_dst,
        batch_dst_neighbor_embedding,
        W_sr, W_sn, W_sl, b_sr.reshape(1, DH), b_sn.reshape(1, DH),
        b_sl.reshape(1, DN),
        W_dr, W_dn, W_dl, b_dr.reshape(1, DH), b_dn.reshape(1, DH),
        b_dl.reshape(1, DN))

    u_src, u_dst = _make_update(MH, B, D2)(
        th_src, th_dst, sidx, didx, new_src, new_dst)
    return (u_src.reshape(M, DN), u_dst.reshape(M, DN))


# speedup vs baseline: 1.1434x; 1.1434x over previous
import functools

import jax
import jax.numpy as jnp
from jax import lax
from jax.experimental import pallas as pl
from jax.experimental.pallas import tpu as pltpu
from jax.experimental.pallas import tpu_sc as plsc

NC, NS, L = 2, 16, 16
NW = NC * NS
CH = 128


def _wid():
    return lax.axis_index("s") * NC + lax.axis_index("c")


@functools.lru_cache(maxsize=None)
def _make_gather(MH, B, D2):
    b_per_w = B // NW
    nch = b_per_w // CH
    mesh = plsc.VectorSubcoreMesh(core_axis_name="c", subcore_axis_name="s")

    @functools.partial(
        pl.kernel, mesh=mesh,
        out_type=(jax.ShapeDtypeStruct((B, D2), jnp.float32),
                  jax.ShapeDtypeStruct((B, D2), jnp.float32)),
        compiler_params=pltpu.CompilerParams(needs_layout_passes=False),
        scratch_types=[pltpu.VMEM((b_per_w,), jnp.int32),
                       pltpu.VMEM((b_per_w, D2), jnp.float32),
                       pltpu.SemaphoreType.DMA],
    )
    def k(src_tab, dst_tab, src_pidx, dst_pidx, out_s, out_d,
          idx_v, rows_v, sem):
        base = _wid() * b_per_w
        for tab, idx, out in ((src_tab, src_pidx, out_s),
                              (dst_tab, dst_pidx, out_d)):
            pltpu.sync_copy(idx.at[pl.ds(base, b_per_w)], idx_v)
            descs = []
            for j in range(nch):
                descs.append(pltpu.async_copy(
                    tab.at[idx_v.at[pl.ds(j * CH, CH)]],
                    rows_v.at[pl.ds(j * CH, CH)], sem))
            for d in descs:
                d.wait()
            pltpu.sync_copy(rows_v, out.at[pl.ds(base, b_per_w)])

    return k


@functools.lru_cache(maxsize=None)
def _make_mlp(B, DN, DG, DH):
    BLK = 2048
    grid = B // BLK
    D2 = 2 * DN

    def body(ids_s, ids_d, gs, ns_, gd, nd_, wsr, wsn, wsl, bsr, bsn, bsl,
             wdr, wdn, wdl, bdr, bdn, bdl, os_, od_):
        f32 = jnp.float32
        i = pl.program_id(0)
        for ids, g2, n, wr, wn, wl, br, bn, bl, out in (
                (ids_s, gs, ns_, wsr, wsn, wsl, bsr, bsn, bsl, os_),
                (ids_d, gd, nd_, wdr, wdn, wdl, bdr, bdn, bdl, od_)):
            par = (ids[pl.ds(i * BLK, BLK)] & 1)[:, None]
            g2v = g2[...]
            g = jnp.where(par == 1, g2v[:, DN:2 * DN], g2v[:, 0:DN])
            wl_top = wl[0:DH, :]
            wl_bot = wl[DH:2 * DH, :]
            Wg = jnp.dot(wr[...], wl_top, preferred_element_type=f32)
            Wn = jnp.dot(wn[...], wl_bot, preferred_element_type=f32)
            b_eff = (jnp.dot(br[...], wl_top, preferred_element_type=f32)
                     + jnp.dot(bn[...], wl_bot, preferred_element_type=f32)
                     + bl[...])
            new = (jnp.dot(g, Wg, preferred_element_type=f32)
                   + jnp.dot(n[...], Wn, preferred_element_type=f32)
                   + b_eff)
            out[...] = jnp.concatenate([new, new], axis=1)

    whole = lambda n: pl.BlockSpec((n,), lambda i: (0,))
    full = lambda shape: pl.BlockSpec(shape, lambda i: (0,) * len(shape))
    row = lambda d: pl.BlockSpec((BLK, d), lambda i: (i, 0))
    return pl.pallas_call(
        body,
        grid=(grid,),
        in_specs=[whole(B), whole(B),
                  row(D2), row(DG), row(D2), row(DG),
                  full((DN, DH)), full((DG, DH)), full((2 * DH, DN)),
                  full((1, DH)), full((1, DH)), full((1, DN)),
                  full((DN, DH)), full((DG, DH)), full((2 * DH, DN)),
                  full((1, DH)), full((1, DH)), full((1, DN))],
        out_specs=[row(D2), row(D2)],
        out_shape=(jax.ShapeDtypeStruct((B, D2), jnp.float32),
                   jax.ShapeDtypeStruct((B, D2), jnp.float32)),
    )


@functools.lru_cache(maxsize=None)
def _make_scatter(MH, B, D2):
    SLABH = (MH // NW) & ~7
    NIT = B // L
    KROWS = 33
    DN = D2 // 2
    QW = DN // L
    mesh = plsc.VectorSubcoreMesh(core_axis_name="c", subcore_axis_name="s")

    @functools.partial(
        pl.kernel, mesh=mesh,
        out_type=(),
        compiler_params=pltpu.CompilerParams(needs_layout_passes=False),
        scratch_types=[pltpu.VMEM((B,), jnp.int32),
                       pltpu.VMEM((KROWS, CH), jnp.int32),
                       pltpu.VMEM((KROWS, CH), jnp.int32),
                       pltpu.VMEM((KROWS * CH,), jnp.int32),
                       pltpu.VMEM((KROWS * CH,), jnp.int32),
                       pltpu.VMEM((CH, D2), jnp.float32),
                       pltpu.VMEM((CH, D2), jnp.float32),
                       pltpu.SemaphoreType.DMA],
    )
    def k(tab_s, tab_d, src_idx, dst_idx, new_src, new_dst,
          ids_v, pair2d_e, pair2d_o, pos_e, pos_o,
          rows_st, vals_st, sem_rw):
        wid = _wid()
        plo = wid * SLABH
        lo = plo * 2
        hi = jnp.where(wid == NW - 1, 2 * MH, lo + SLABH * 2)

        iota = lax.iota(jnp.int32, L)
        one = jnp.int32(1)
        zero = jnp.int32(0)

        def splat_last(x):
            return plsc.cummax(lax.rev(plsc.cummax(x), (0,)))

        def scan_ids(idx_hbm):
            pltpu.sync_copy(idx_hbm, ids_v)

            def step(i, carry):
                cnt_e, cnt_o = carry
                v = ids_v[pl.ds(i * L, L)]
                posv = i * L + iota
                inb = (v >= lo) & (v < hi)
                pair = v >> 1
                odd = (v & 1) == 1
                m_e = inb & jnp.logical_not(odd)
                m_o = inb & odd
                for m, cnt, pair2d, pos in ((m_e, cnt_e, pair2d_e, pos_e),
                                            (m_o, cnt_o, pair2d_o, pos_o)):
                    incl = plsc.cumsum(jnp.where(m, one, zero))
                    tgt = jnp.maximum(cnt + incl - 1, 0)
                    plsc.store_scatter(pair2d, [tgt >> 7, tgt & (CH - 1)],
                                       pair, mask=m)
                    plsc.store_scatter(pos, [tgt], posv, mask=m)
                return (cnt_e + plsc.all_reduce_population_count(m_e),
                        cnt_o + plsc.all_reduce_population_count(m_o))

            zeros = jnp.zeros((L,), jnp.int32)
            cnt_e, cnt_o = lax.fori_loop(0, NIT, step, (zeros, zeros))

            def finish(cnt_v, pair2d, pos):
                n = cnt_v[L - 1]
                n_pad = ((n + CH - 1) // CH) * CH
                j = jnp.maximum(n - 1, 0)
                vrow = pair2d[j >> 7, pl.ds((j & (CH - 1)) & ~(L - 1), L)]
                vpos = pos[pl.ds(j & ~(L - 1), L)]
                lane = j & (L - 1)
                pad_pair = splat_last(jnp.where(iota == lane, vrow, -1))
                pad_pos = splat_last(jnp.where(iota == lane, vpos, -1))

                def pad_step(t, _):
                    p = n + t * L + iota
                    plsc.store_scatter(pair2d, [p >> 7, p & (CH - 1)],
                                       pad_pair)
                    plsc.store_scatter(pos, [p], pad_pos)
                    return 0

                lax.fori_loop(0, (n_pad - n + L - 1) // L, pad_step, 0)
                return n_pad

            return (finish(cnt_e, pair2d_e, pos_e),
                    finish(cnt_o, pair2d_o, pos_o))

        def apply_pass(new_hbm, out_hbm, pair2d, pos, n_pad, off):
            def chunk(c, _):
                pltpu.async_copy(out_hbm.at[pair2d.at[c]], rows_st,
                                 sem_rw).wait()
                pltpu.async_copy(new_hbm.at[pos.at[pl.ds(c * CH, CH)]],
                                 vals_st, sem_rw).wait()

                def merge(r, _):
                    for q in range(QW):
                        rows_st[r, pl.ds(off + q * L, L)] = \
                            vals_st[r, pl.ds(q * L, L)]
                    return 0

                lax.fori_loop(0, CH, merge, 0)
                pltpu.async_copy(rows_st, out_hbm.at[pair2d.at[c]],
                                 sem_rw).wait()
                return 0

            lax.fori_loop(0, n_pad // CH, chunk, 0)

        ne, no = scan_ids(src_idx)
        apply_pass(new_src, tab_s, pair2d_e, pos_e, ne, 0)
        apply_pass(new_src, tab_s, pair2d_o, pos_o, no, DN)
        ne, no = scan_ids(dst_idx)
        apply_pass(new_dst, tab_d, pair2d_e, pos_e, ne, 0)
        apply_pass(new_dst, tab_d, pair2d_o, pos_o, no, DN)

    return k


def kernel(src_node_ids, dst_node_ids, src_previous_embedding,
           dst_previous_embedding, batch_src_neighbor_embedding,
           batch_dst_neighbor_embedding, W_sn, b_sn, W_sr, b_sr, W_sl, b_sl,
           W_dn, b_dn, W_dr, b_dr, W_dl, b_dl):
    M, DN = src_previous_embedding.shape
    B = src_node_ids.shape[0]
    DG = batch_src_neighbor_embedding.shape[1]
    DH = W_sn.shape[1]
    MH, D2 = M // 2, 2 * DN

    sidx = src_node_ids.astype(jnp.int32)
    didx = dst_node_ids.astype(jnp.int32)
    th_src = src_previous_embedding.reshape(MH, D2)
    th_dst = dst_previous_embedding.reshape(MH, D2)

    g2_src, g2_dst = _make_gather(MH, B, D2)(th_src, th_dst,
                                             sidx >> 1, didx >> 1)

    new_src, new_dst = _make_mlp(B, DN, DG, DH)(
        sidx, didx, g2_src, batch_src_neighbor_embedding, g2_dst,
        batch_dst_neighbor_embedding,
        W_sr, W_sn, W_sl, b_sr.reshape(1, DH), b_sn.reshape(1, DH),
        b_sl.reshape(1, DN),
        W_dr, W_dn, W_dl, b_dr.reshape(1, DH), b_dn.reshape(1, DH),
        b_dl.reshape(1, DN))

    r_s = jax.new_ref(th_src)
    r_d = jax.new_ref(th_dst)
    _make_scatter(MH, B, D2)(r_s, r_d, sidx, didx, new_src, new_dst)
    return (r_s[...].reshape(M, DN), r_d[...].reshape(M, DN))
